# Initial kernel scaffold; baseline (speedup 1.0000x reference)
import functools

import jax
import jax.numpy as jnp
from jax import lax
from jax.experimental import pallas as pl
from jax.experimental.pallas import tpu as pltpu
from jax.experimental.pallas import tpu_sc as plsc

N = 10000
E = 320000
D = 128

NC = 2
NS = 16
NW = NC * NS
EPT = E // NW
CH = 80
NCH = EPT // CH
RPT = N // NS
RSTG = 125
NP = 10240
BLK = 1024
GRID = NP // BLK

_mesh = plsc.VectorSubcoreMesh(core_axis_name="c", subcore_axis_name="s")


@functools.partial(
    pl.kernel,
    out_type=jax.ShapeDtypeStruct((NW, NP), jnp.float32),
    mesh=_mesh,
    scratch_types=[
        pltpu.VMEM((N,), jnp.float32),
        pltpu.VMEM((EPT,), jnp.int32),
    ],
)
def _sc_degree(dst_hbm, out_hbm, acc_v, idx_v):
    c = lax.axis_index("c")
    s = lax.axis_index("s")
    wid = c * NS + s

    def zero(i, carry):
        acc_v[pl.ds(i * 16, 16)] = jnp.zeros((16,), jnp.float32)
        return carry

    lax.fori_loop(0, N // 16, zero, 0)

    pltpu.sync_copy(dst_hbm.at[pl.ds(wid * EPT, EPT)], idx_v)
    ones = jnp.ones((16,), jnp.float32)

    def count(i, carry):
        idx = idx_v[pl.ds(i * 16, 16)]
        plsc.addupdate_scatter(acc_v, [idx], ones)
        return carry

    lax.fori_loop(0, EPT // 16, count, 0)
    pltpu.sync_copy(acc_v, out_hbm.at[wid, pl.ds(0, N)])


@functools.partial(
    pl.kernel,
    out_type=jax.ShapeDtypeStruct((NC, N, D), jnp.float32),
    mesh=_mesh,
    scratch_types=[
        pltpu.VMEM((2, CH), jnp.int32),
        pltpu.VMEM((CH, D), jnp.float32),
        pltpu.VMEM((RSTG, D), jnp.float32),
        pltpu.VMEM_SHARED((N, D), jnp.float32),
        pltpu.SemaphoreType.DMA,
    ],
)
def _sc_scatter(hp_hbm, src_hbm, dst_hbm, out_hbm, idx_v, rows_v, stage_v, s_sh, sem):
    c = lax.axis_index("c")
    s = lax.axis_index("s")
    wid = c * NS + s

    def zero(i, carry):
        stage_v[i // 8, pl.ds((i % 8) * 16, 16)] = jnp.zeros((16,), jnp.float32)
        return carry

    lax.fori_loop(0, RSTG * 8, zero, 0)

    def zcopy(k, carry):
        pltpu.sync_copy(stage_v, s_sh.at[pl.ds(s * RPT + k * RSTG, RSTG)])
        return carry

    lax.fori_loop(0, RPT // RSTG, zcopy, 0)
    plsc.subcore_barrier()

    ebase = wid * EPT

    def chunk(i, carry):
        pltpu.sync_copy(src_hbm.at[pl.ds(ebase + i * CH, CH)], idx_v.at[0])
        pltpu.sync_copy(dst_hbm.at[pl.ds(ebase + i * CH, CH)], idx_v.at[1])
        pltpu.async_copy(hp_hbm.at[idx_v.at[0]], rows_v, sem).wait()
        pltpu.sync_copy(rows_v, s_sh.at[idx_v.at[1]], add=True)
        return carry

    lax.fori_loop(0, NCH, chunk, 0)
    plsc.subcore_barrier()

    def out_copy(k, carry):
        r0 = s * RPT + k * RSTG
        pltpu.sync_copy(s_sh.at[pl.ds(r0, RSTG)], stage_v)
        pltpu.sync_copy(stage_v, out_hbm.at[c, pl.ds(r0, RSTG)])
        return carry

    lax.fori_loop(0, RPT // RSTG, out_copy, 0)


def _dinv(dp_block):
    deg = jnp.sum(dp_block, axis=0) + 1.0
    return lax.rsqrt(jnp.maximum(deg, 1.0))


def _mm1_body(x_ref, w_ref, dp_ref, o_ref):
    dinv = _dinv(dp_ref[...])
    h = jnp.dot(x_ref[...], w_ref[...], preferred_element_type=jnp.float32)
    o_ref[...] = h * dinv[:, None]


_mm1 = pl.pallas_call(
    _mm1_body,
    grid=(GRID,),
    in_specs=[
        pl.BlockSpec((BLK, D), lambda i: (i, 0)),
        pl.BlockSpec((D, D), lambda i: (0, 0)),
        pl.BlockSpec((NW, BLK), lambda i: (0, i)),
    ],
    out_specs=pl.BlockSpec((BLK, D), lambda i: (i, 0)),
    out_shape=jax.ShapeDtypeStruct((N, D), jnp.float32),
)


def _mm2_body(s_ref, hp_ref, dp_ref, b_ref, w_ref, o_ref):
    dinv = _dinv(dp_ref[...])
    tot = s_ref[0] + s_ref[1] + hp_ref[...]
    z = jnp.maximum(tot * dinv[:, None] + b_ref[...], 0.0)
    h = jnp.dot(z, w_ref[...], preferred_element_type=jnp.float32)
    o_ref[...] = h * dinv[:, None]


_mm2 = pl.pallas_call(
    _mm2_body,
    grid=(GRID,),
    in_specs=[
        pl.BlockSpec((NC, BLK, D), lambda i: (0, i, 0)),
        pl.BlockSpec((BLK, D), lambda i: (i, 0)),
        pl.BlockSpec((NW, BLK), lambda i: (0, i)),
        pl.BlockSpec((1, D), lambda i: (0, 0)),
        pl.BlockSpec((D, D), lambda i: (0, 0)),
    ],
    out_specs=pl.BlockSpec((BLK, D), lambda i: (i, 0)),
    out_shape=jax.ShapeDtypeStruct((N, D), jnp.float32),
)


def _fin_body(s_ref, hp_ref, dp_ref, b_ref, o_ref):
    dinv = _dinv(dp_ref[...])
    tot = s_ref[0] + s_ref[1] + hp_ref[...]
    o_ref[...] = tot * dinv[:, None] + b_ref[...]


_fin = pl.pallas_call(
    _fin_body,
    grid=(GRID,),
    in_specs=[
        pl.BlockSpec((NC, BLK, D), lambda i: (0, i, 0)),
        pl.BlockSpec((BLK, D), lambda i: (i, 0)),
        pl.BlockSpec((NW, BLK), lambda i: (0, i)),
        pl.BlockSpec((1, D), lambda i: (0, 0)),
    ],
    out_specs=pl.BlockSpec((BLK, D), lambda i: (i, 0)),
    out_shape=jax.ShapeDtypeStruct((N, D), jnp.float32),
)


def kernel(x, edge_index, W1, b1, W2, b2):
    src = edge_index[0]
    dst = edge_index[1]
    degp = _sc_degree(dst)
    h1p = _mm1(x, W1, degp)
    s1 = _sc_scatter(h1p, src, dst)
    h2p = _mm2(s1, h1p, degp, b1.reshape(1, D), W2)
    s2 = _sc_scatter(h2p, src, dst)
    return _fin(s2, h2p, degp, b2.reshape(1, D))


# speedup vs baseline: 14.2805x; 14.2805x over previous
import functools

import jax
import jax.numpy as jnp
from jax import lax
from jax.experimental import pallas as pl
from jax.experimental.pallas import tpu as pltpu
from jax.experimental.pallas import tpu_sc as plsc

N = 10000
E = 320000
D = 128

NC = 2
NS = 16
NW = NC * NS
EPT = E // NW
CH = 80
NCH = EPT // CH
NP = 10240
RPT = NP // NS
RSTG = 128
BLK = 1024
GRID = NP // BLK

_mesh = plsc.VectorSubcoreMesh(core_axis_name="c", subcore_axis_name="s")


@functools.partial(
    pl.kernel,
    out_type=jax.ShapeDtypeStruct((NW * NP,), jnp.float32),
    mesh=_mesh,
    scratch_types=[
        pltpu.VMEM((NP,), jnp.float32),
        pltpu.VMEM((EPT,), jnp.int32),
    ],
    compiler_params=pltpu.CompilerParams(needs_layout_passes=False),
)
def _sc_degree(dst_hbm, out_hbm, acc_v, idx_v):
    c = lax.axis_index("c")
    s = lax.axis_index("s")
    wid = c * NS + s

    def zero(i, carry):
        acc_v[pl.ds(i * 16, 16)] = jnp.zeros((16,), jnp.float32)
        return carry

    lax.fori_loop(0, NP // 16, zero, 0)

    pltpu.sync_copy(dst_hbm.at[pl.ds(wid * EPT, EPT)], idx_v)
    ones = jnp.ones((16,), jnp.float32)

    def count(i, carry):
        idx = idx_v[pl.ds(i * 16, 16)]
        plsc.addupdate_scatter(acc_v, [idx], ones)
        return carry

    lax.fori_loop(0, EPT // 16, count, 0)
    pltpu.sync_copy(acc_v, out_hbm.at[pl.ds(wid * NP, NP)])


@functools.partial(
    pl.kernel,
    out_type=jax.ShapeDtypeStruct((NC, NP, D), jnp.float32),
    mesh=_mesh,
    scratch_types=[
        pltpu.VMEM((2, CH), jnp.int32),
        pltpu.VMEM((CH, D), jnp.float32),
        pltpu.VMEM((RSTG, D), jnp.float32),
        pltpu.VMEM_SHARED((NP, D), jnp.float32),
        pltpu.SemaphoreType.DMA,
    ],
)
def _sc_scatter(hp_hbm, src_hbm, dst_hbm, out_hbm, idx_v, rows_v, stage_v, s_sh, sem):
    c = lax.axis_index("c")
    s = lax.axis_index("s")
    wid = c * NS + s

    def zero(i, carry):
        stage_v[i // 8, pl.ds((i % 8) * 16, 16)] = jnp.zeros((16,), jnp.float32)
        return carry

    lax.fori_loop(0, RSTG * 8, zero, 0)

    def zcopy(k, carry):
        pltpu.sync_copy(stage_v, s_sh.at[pl.ds(s * RPT + k * RSTG, RSTG)])
        return carry

    lax.fori_loop(0, RPT // RSTG, zcopy, 0)
    plsc.subcore_barrier()

    ebase = wid * EPT

    def chunk(i, carry):
        pltpu.sync_copy(src_hbm.at[pl.ds(ebase + i * CH, CH)], idx_v.at[0])
        pltpu.sync_copy(dst_hbm.at[pl.ds(ebase + i * CH, CH)], idx_v.at[1])
        pltpu.async_copy(hp_hbm.at[idx_v.at[0]], rows_v, sem).wait()
        pltpu.sync_copy(rows_v, s_sh.at[idx_v.at[1]], add=True)
        return carry

    lax.fori_loop(0, NCH, chunk, 0)
    plsc.subcore_barrier()

    def out_copy(k, carry):
        r0 = s * RPT + k * RSTG
        pltpu.sync_copy(s_sh.at[pl.ds(r0, RSTG)], stage_v)
        pltpu.sync_copy(stage_v, out_hbm.at[c, pl.ds(r0, RSTG)])
        return carry

    lax.fori_loop(0, RPT // RSTG, out_copy, 0)


def _dinv(dp_block):
    deg = jnp.sum(dp_block, axis=0) + 1.0
    return lax.rsqrt(jnp.maximum(deg, 1.0))


def _mm1_body(x_ref, w_ref, dp_ref, o_ref):
    dinv = _dinv(dp_ref[...])
    h = jnp.dot(x_ref[...], w_ref[...], preferred_element_type=jnp.float32)
    o_ref[...] = h * dinv[:, None]


_mm1 = pl.pallas_call(
    _mm1_body,
    grid=(GRID,),
    in_specs=[
        pl.BlockSpec((BLK, D), lambda i: (i, 0)),
        pl.BlockSpec((D, D), lambda i: (0, 0)),
        pl.BlockSpec((NW, BLK), lambda i: (0, i)),
    ],
    out_specs=pl.BlockSpec((BLK, D), lambda i: (i, 0)),
    out_shape=jax.ShapeDtypeStruct((N, D), jnp.float32),
)


def _mm2_body(s_ref, hp_ref, dp_ref, b_ref, w_ref, o_ref):
    dinv = _dinv(dp_ref[...])
    tot = s_ref[0] + s_ref[1] + hp_ref[...]
    z = jnp.maximum(tot * dinv[:, None] + b_ref[...], 0.0)
    h = jnp.dot(z, w_ref[...], preferred_element_type=jnp.float32)
    o_ref[...] = h * dinv[:, None]


_mm2 = pl.pallas_call(
    _mm2_body,
    grid=(GRID,),
    in_specs=[
        pl.BlockSpec((NC, BLK, D), lambda i: (0, i, 0)),
        pl.BlockSpec((BLK, D), lambda i: (i, 0)),
        pl.BlockSpec((NW, BLK), lambda i: (0, i)),
        pl.BlockSpec((1, D), lambda i: (0, 0)),
        pl.BlockSpec((D, D), lambda i: (0, 0)),
    ],
    out_specs=pl.BlockSpec((BLK, D), lambda i: (i, 0)),
    out_shape=jax.ShapeDtypeStruct((N, D), jnp.float32),
)


def _fin_body(s_ref, hp_ref, dp_ref, b_ref, o_ref):
    dinv = _dinv(dp_ref[...])
    tot = s_ref[0] + s_ref[1] + hp_ref[...]
    o_ref[...] = tot * dinv[:, None] + b_ref[...]


_fin = pl.pallas_call(
    _fin_body,
    grid=(GRID,),
    in_specs=[
        pl.BlockSpec((NC, BLK, D), lambda i: (0, i, 0)),
        pl.BlockSpec((BLK, D), lambda i: (i, 0)),
        pl.BlockSpec((NW, BLK), lambda i: (0, i)),
        pl.BlockSpec((1, D), lambda i: (0, 0)),
    ],
    out_specs=pl.BlockSpec((BLK, D), lambda i: (i, 0)),
    out_shape=jax.ShapeDtypeStruct((N, D), jnp.float32),
)


def kernel(x, edge_index, W1, b1, W2, b2):
    src = edge_index[0]
    dst = edge_index[1]
    degp = _sc_degree(dst).reshape(NW, NP)
    h1p = _mm1(x, W1, degp)
    s1 = _sc_scatter(h1p, src, dst)
    h2p = _mm2(s1, h1p, degp, b1.reshape(1, D), W2)
    s2 = _sc_scatter(h2p, src, dst)
    return _fin(s2, h2p, degp, b2.reshape(1, D))


# speedup vs baseline: 25.7911x; 1.8060x over previous
import functools

import jax
import jax.numpy as jnp
from jax import lax
from jax.experimental import pallas as pl
from jax.experimental.pallas import tpu as pltpu
from jax.experimental.pallas import tpu_sc as plsc

N = 10000
E = 320000
D = 128

NC = 2
NS = 16
NW = NC * NS
EPT = E // NW
CH = 80
NCH = EPT // CH
NP = 10240
RPT = NP // NS
RSTG = 128
BLK = 1024
GRID = NP // BLK

_mesh = plsc.VectorSubcoreMesh(core_axis_name="c", subcore_axis_name="s")


@functools.partial(
    pl.kernel,
    out_type=jax.ShapeDtypeStruct((NW * NP,), jnp.float32),
    mesh=_mesh,
    scratch_types=[
        pltpu.VMEM((NP,), jnp.float32),
        pltpu.VMEM((EPT,), jnp.int32),
    ],
    compiler_params=pltpu.CompilerParams(needs_layout_passes=False),
)
def _sc_degree(dst_hbm, out_hbm, acc_v, idx_v):
    c = lax.axis_index("c")
    s = lax.axis_index("s")
    wid = c * NS + s

    def zero(i, carry):
        acc_v[pl.ds(i * 16, 16)] = jnp.zeros((16,), jnp.float32)
        return carry

    lax.fori_loop(0, NP // 16, zero, 0)

    pltpu.sync_copy(dst_hbm.at[pl.ds(wid * EPT, EPT)], idx_v)
    ones = jnp.ones((16,), jnp.float32)

    def count(i, carry):
        idx = idx_v[pl.ds(i * 16, 16)]
        plsc.addupdate_scatter(acc_v, [idx], ones)
        return carry

    lax.fori_loop(0, EPT // 16, count, 0)
    pltpu.sync_copy(acc_v, out_hbm.at[pl.ds(wid * NP, NP)])


NSUP = (NCH - 1) // 2


@functools.partial(
    pl.kernel,
    out_type=jax.ShapeDtypeStruct((NC, NP, D), jnp.float32),
    mesh=_mesh,
    scratch_types=[
        pltpu.VMEM((NCH, CH), jnp.int32),
        pltpu.VMEM((2, CH), jnp.int32),
        pltpu.VMEM((2, CH, D), jnp.float32),
        pltpu.VMEM_SHARED((NP, D), jnp.float32),
        [pltpu.SemaphoreType.DMA] * 2,
        [pltpu.SemaphoreType.DMA] * 2,
        [pltpu.SemaphoreType.DMA] * 2,
    ],
)
def _sc_scatter(hp_hbm, src_hbm, dst3_hbm, out_hbm, didx_v, sidx_v, rows_v,
                s_sh, sem_i, sem_g, sem_s):
    c = lax.axis_index("c")
    s = lax.axis_index("s")
    wid = c * NS + s
    ebase = wid * EPT

    pltpu.sync_copy(dst3_hbm.at[wid], didx_v)
    pltpu.sync_copy(src_hbm.at[pl.ds(ebase, CH)], sidx_v.at[0])
    pltpu.sync_copy(src_hbm.at[pl.ds(ebase + CH, CH)], sidx_v.at[1])

    def zero(i, carry):
        rows_v[0, i // 8, pl.ds((i % 8) * 16, 16)] = jnp.zeros((16,), jnp.float32)
        return carry

    lax.fori_loop(0, CH * 8, zero, 0)

    def zcopy(k, carry):
        pltpu.sync_copy(rows_v.at[0], s_sh.at[pl.ds(s * RPT + k * CH, CH)])
        return carry

    lax.fori_loop(0, RPT // CH, zcopy, 0)

    pltpu.async_copy(hp_hbm.at[sidx_v.at[0]], rows_v.at[0], sem_g[0])
    pltpu.async_copy(hp_hbm.at[sidx_v.at[1]], rows_v.at[1], sem_g[1])
    plsc.subcore_barrier()

    def super_iter(si, carry):
        i0 = si * 2
        for b in range(2):
            i = i0 + b
            nxt = i + 2
            pltpu.make_async_copy(
                hp_hbm.at[sidx_v.at[b]], rows_v.at[b], sem_g[b]).wait()

            @pl.when(nxt < NCH)
            def _():
                pltpu.async_copy(
                    src_hbm.at[pl.ds(ebase + nxt * CH, CH)], sidx_v.at[b],
                    sem_i[b])

            pltpu.async_copy(
                rows_v.at[b], s_sh.at[didx_v.at[i]], sem_s[b], add=True)
        for b in range(2):
            nxt = i0 + 2 + b
            pltpu.make_async_copy(
                rows_v.at[b], s_sh.at[didx_v.at[0]], sem_s[b]).wait()

            @pl.when(nxt < NCH)
            def _():
                pltpu.make_async_copy(
                    src_hbm.at[pl.ds(ebase, CH)], sidx_v.at[b], sem_i[b]).wait()
                pltpu.async_copy(
                    hp_hbm.at[sidx_v.at[b]], rows_v.at[b], sem_g[b])

        return carry

    lax.fori_loop(0, NSUP, super_iter, 0)

    pltpu.make_async_copy(hp_hbm.at[sidx_v.at[0]], rows_v.at[0], sem_g[0]).wait()
    pltpu.async_copy(rows_v.at[0], s_sh.at[didx_v.at[NCH - 1]], sem_s[0], add=True)
    pltpu.make_async_copy(rows_v.at[0], s_sh.at[didx_v.at[0]], sem_s[0]).wait()
    plsc.subcore_barrier()

    def out_copy(k, carry):
        r0 = s * RPT + k * CH
        pltpu.sync_copy(s_sh.at[pl.ds(r0, CH)], rows_v.at[0])
        pltpu.sync_copy(rows_v.at[0], out_hbm.at[c, pl.ds(r0, CH)])
        return carry

    lax.fori_loop(0, RPT // CH, out_copy, 0)


def _dinv(dp_block):
    deg = jnp.sum(dp_block, axis=0) + 1.0
    return lax.rsqrt(jnp.maximum(deg, 1.0))


def _mm1_body(x_ref, w_ref, dp_ref, o_ref):
    dinv = _dinv(dp_ref[...])
    h = jnp.dot(x_ref[...], w_ref[...], preferred_element_type=jnp.float32)
    o_ref[...] = h * dinv[:, None]


_mm1 = pl.pallas_call(
    _mm1_body,
    grid=(GRID,),
    in_specs=[
        pl.BlockSpec((BLK, D), lambda i: (i, 0)),
        pl.BlockSpec((D, D), lambda i: (0, 0)),
        pl.BlockSpec((NW, BLK), lambda i: (0, i)),
    ],
    out_specs=pl.BlockSpec((BLK, D), lambda i: (i, 0)),
    out_shape=jax.ShapeDtypeStruct((N, D), jnp.float32),
)


def _mm2_body(s_ref, hp_ref, dp_ref, b_ref, w_ref, o_ref):
    dinv = _dinv(dp_ref[...])
    tot = s_ref[0] + s_ref[1] + hp_ref[...]
    z = jnp.maximum(tot * dinv[:, None] + b_ref[...], 0.0)
    h = jnp.dot(z, w_ref[...], preferred_element_type=jnp.float32)
    o_ref[...] = h * dinv[:, None]


_mm2 = pl.pallas_call(
    _mm2_body,
    grid=(GRID,),
    in_specs=[
        pl.BlockSpec((NC, BLK, D), lambda i: (0, i, 0)),
        pl.BlockSpec((BLK, D), lambda i: (i, 0)),
        pl.BlockSpec((NW, BLK), lambda i: (0, i)),
        pl.BlockSpec((1, D), lambda i: (0, 0)),
        pl.BlockSpec((D, D), lambda i: (0, 0)),
    ],
    out_specs=pl.BlockSpec((BLK, D), lambda i: (i, 0)),
    out_shape=jax.ShapeDtypeStruct((N, D), jnp.float32),
)


def _fin_body(s_ref, hp_ref, dp_ref, b_ref, o_ref):
    dinv = _dinv(dp_ref[...])
    tot = s_ref[0] + s_ref[1] + hp_ref[...]
    o_ref[...] = tot * dinv[:, None] + b_ref[...]


_fin = pl.pallas_call(
    _fin_body,
    grid=(GRID,),
    in_specs=[
        pl.BlockSpec((NC, BLK, D), lambda i: (0, i, 0)),
        pl.BlockSpec((BLK, D), lambda i: (i, 0)),
        pl.BlockSpec((NW, BLK), lambda i: (0, i)),
        pl.BlockSpec((1, D), lambda i: (0, 0)),
    ],
    out_specs=pl.BlockSpec((BLK, D), lambda i: (i, 0)),
    out_shape=jax.ShapeDtypeStruct((N, D), jnp.float32),
)


def kernel(x, edge_index, W1, b1, W2, b2):
    src = edge_index[0]
    dst = edge_index[1]
    dst3 = dst.reshape(NW, NCH, CH)
    degp = _sc_degree(dst).reshape(NW, NP)
    h1p = _mm1(x, W1, degp)
    s1 = _sc_scatter(h1p, src, dst3)
    h2p = _mm2(s1, h1p, degp, b1.reshape(1, D), W2)
    s2 = _sc_scatter(h2p, src, dst3)
    return _fin(s2, h2p, degp, b2.reshape(1, D))


# speedup vs baseline: 26.7022x; 1.0353x over previous
import functools

import jax
import jax.numpy as jnp
from jax import lax
from jax.experimental import pallas as pl
from jax.experimental.pallas import tpu as pltpu
from jax.experimental.pallas import tpu_sc as plsc

N = 10000
E = 320000
D = 128

NC = 2
NS = 16
NW = NC * NS
EPT = E // NW
CH = 100
NCH = EPT // CH
NP = 10240
RPT = NP // NS
RSTG = 80
BLK = 1024
GRID = NP // BLK

_mesh = plsc.VectorSubcoreMesh(core_axis_name="c", subcore_axis_name="s")


@functools.partial(
    pl.kernel,
    out_type=jax.ShapeDtypeStruct((NW * NP,), jnp.float32),
    mesh=_mesh,
    scratch_types=[
        pltpu.VMEM((NP,), jnp.float32),
        pltpu.VMEM((EPT,), jnp.int32),
    ],
    compiler_params=pltpu.CompilerParams(needs_layout_passes=False),
)
def _sc_degree(dst_hbm, out_hbm, acc_v, idx_v):
    c = lax.axis_index("c")
    s = lax.axis_index("s")
    wid = c * NS + s

    def zero(i, carry):
        acc_v[pl.ds(i * 16, 16)] = jnp.zeros((16,), jnp.float32)
        return carry

    lax.fori_loop(0, NP // 16, zero, 0)

    pltpu.sync_copy(dst_hbm.at[pl.ds(wid * EPT, EPT)], idx_v)
    ones = jnp.ones((16,), jnp.float32)

    def count(i, carry):
        idx = idx_v[pl.ds(i * 16, 16)]
        plsc.addupdate_scatter(acc_v, [idx], ones)
        return carry

    lax.fori_loop(0, EPT // 16, count, 0)
    pltpu.sync_copy(acc_v, out_hbm.at[pl.ds(wid * NP, NP)])


NSUP = NCH // 2


@functools.partial(
    pl.kernel,
    out_type=jax.ShapeDtypeStruct((NC, NP, D), jnp.float32),
    mesh=_mesh,
    scratch_types=[
        pltpu.VMEM((NCH, CH), jnp.int32),
        pltpu.VMEM((2, 1, CH), jnp.int32),
        pltpu.VMEM((2, CH, D), jnp.float32),
        pltpu.VMEM_SHARED((NP, D), jnp.float32),
        [pltpu.SemaphoreType.DMA] * 2,
        [pltpu.SemaphoreType.DMA] * 2,
        [pltpu.SemaphoreType.DMA] * 2,
    ],
)
def _sc_scatter(hp_hbm, src4_hbm, dst3_hbm, out_hbm, didx_v, sidx_v, rows_v,
                s_sh, sem_i, sem_g, sem_s):
    c = lax.axis_index("c")
    s = lax.axis_index("s")
    wid = c * NS + s

    pltpu.sync_copy(dst3_hbm.at[wid], didx_v)
    pltpu.sync_copy(src4_hbm.at[wid, 0], sidx_v.at[0])
    pltpu.sync_copy(src4_hbm.at[wid, 1], sidx_v.at[1])

    def zero(i, carry):
        rows_v[0, i // 8, pl.ds((i % 8) * 16, 16)] = jnp.zeros((16,), jnp.float32)
        return carry

    lax.fori_loop(0, RSTG * 8, zero, 0)

    def zcopy(k, carry):
        pltpu.sync_copy(rows_v.at[0, pl.ds(0, RSTG)],
                        s_sh.at[pl.ds(s * RPT + k * RSTG, RSTG)])
        return carry

    lax.fori_loop(0, RPT // RSTG, zcopy, 0)

    pltpu.async_copy(hp_hbm.at[sidx_v.at[0, 0]], rows_v.at[0], sem_g[0])
    pltpu.async_copy(hp_hbm.at[sidx_v.at[1, 0]], rows_v.at[1], sem_g[1])
    plsc.subcore_barrier()

    def super_iter(si, carry):
        i0 = si * 2
        for b in range(2):
            i = i0 + b
            nxt = i + 2
            pltpu.make_async_copy(
                hp_hbm.at[sidx_v.at[b, 0]], rows_v.at[b], sem_g[b]).wait()

            @pl.when(nxt < NCH)
            def _():
                pltpu.async_copy(
                    src4_hbm.at[wid, nxt], sidx_v.at[b], sem_i[b])

            pltpu.async_copy(
                rows_v.at[b], s_sh.at[didx_v.at[i]], sem_s[b], add=True)
        for b in range(2):
            nxt = i0 + 2 + b
            pltpu.make_async_copy(
                rows_v.at[b], s_sh.at[didx_v.at[0]], sem_s[b]).wait()

            @pl.when(nxt < NCH)
            def _():
                pltpu.make_async_copy(
                    src4_hbm.at[wid, 0], sidx_v.at[b], sem_i[b]).wait()
                pltpu.async_copy(
                    hp_hbm.at[sidx_v.at[b, 0]], rows_v.at[b], sem_g[b])

        return carry

    lax.fori_loop(0, NSUP, super_iter, 0)
    plsc.subcore_barrier()

    def out_copy(k, carry):
        r0 = s * RPT + k * RSTG
        pltpu.sync_copy(s_sh.at[pl.ds(r0, RSTG)], rows_v.at[0, pl.ds(0, RSTG)])
        pltpu.sync_copy(rows_v.at[0, pl.ds(0, RSTG)], out_hbm.at[c, pl.ds(r0, RSTG)])
        return carry

    lax.fori_loop(0, RPT // RSTG, out_copy, 0)


def _dinv(dp_block):
    deg = jnp.sum(dp_block, axis=0) + 1.0
    return lax.rsqrt(jnp.maximum(deg, 1.0))


def _mm1_body(x_ref, w_ref, dp_ref, o_ref):
    dinv = _dinv(dp_ref[...])
    h = jnp.dot(x_ref[...], w_ref[...], preferred_element_type=jnp.float32)
    o_ref[...] = h * dinv[:, None]


_mm1 = pl.pallas_call(
    _mm1_body,
    grid=(GRID,),
    in_specs=[
        pl.BlockSpec((BLK, D), lambda i: (i, 0)),
        pl.BlockSpec((D, D), lambda i: (0, 0)),
        pl.BlockSpec((NW, BLK), lambda i: (0, i)),
    ],
    out_specs=pl.BlockSpec((BLK, D), lambda i: (i, 0)),
    out_shape=jax.ShapeDtypeStruct((N, D), jnp.float32),
)


def _mm2_body(s_ref, hp_ref, dp_ref, b_ref, w_ref, o_ref):
    dinv = _dinv(dp_ref[...])
    tot = s_ref[0] + s_ref[1] + hp_ref[...]
    z = jnp.maximum(tot * dinv[:, None] + b_ref[...], 0.0)
    h = jnp.dot(z, w_ref[...], preferred_element_type=jnp.float32)
    o_ref[...] = h * dinv[:, None]


_mm2 = pl.pallas_call(
    _mm2_body,
    grid=(GRID,),
    in_specs=[
        pl.BlockSpec((NC, BLK, D), lambda i: (0, i, 0)),
        pl.BlockSpec((BLK, D), lambda i: (i, 0)),
        pl.BlockSpec((NW, BLK), lambda i: (0, i)),
        pl.BlockSpec((1, D), lambda i: (0, 0)),
        pl.BlockSpec((D, D), lambda i: (0, 0)),
    ],
    out_specs=pl.BlockSpec((BLK, D), lambda i: (i, 0)),
    out_shape=jax.ShapeDtypeStruct((N, D), jnp.float32),
)


def _fin_body(s_ref, hp_ref, dp_ref, b_ref, o_ref):
    dinv = _dinv(dp_ref[...])
    tot = s_ref[0] + s_ref[1] + hp_ref[...]
    o_ref[...] = tot * dinv[:, None] + b_ref[...]


_fin = pl.pallas_call(
    _fin_body,
    grid=(GRID,),
    in_specs=[
        pl.BlockSpec((NC, BLK, D), lambda i: (0, i, 0)),
        pl.BlockSpec((BLK, D), lambda i: (i, 0)),
        pl.BlockSpec((NW, BLK), lambda i: (0, i)),
        pl.BlockSpec((1, D), lambda i: (0, 0)),
    ],
    out_specs=pl.BlockSpec((BLK, D), lambda i: (i, 0)),
    out_shape=jax.ShapeDtypeStruct((N, D), jnp.float32),
)


def kernel(x, edge_index, W1, b1, W2, b2):
    src = edge_index[0]
    dst = edge_index[1]
    src4 = src.reshape(NW, NCH, 1, CH)
    dst3 = dst.reshape(NW, NCH, CH)
    degp = _sc_degree(dst).reshape(NW, NP)
    h1p = _mm1(x, W1, degp)
    s1 = _sc_scatter(h1p, src4, dst3)
    h2p = _mm2(s1, h1p, degp, b1.reshape(1, D), W2)
    s2 = _sc_scatter(h2p, src4, dst3)
    return _fin(s2, h2p, degp, b2.reshape(1, D))


# speedup vs baseline: 27.9162x; 1.0455x over previous
import functools

import jax
import jax.numpy as jnp
from jax import lax
from jax.experimental import pallas as pl
from jax.experimental.pallas import tpu as pltpu
from jax.experimental.pallas import tpu_sc as plsc

N = 10000
E = 320000
D = 128

NC = 2
NS = 16
NW = NC * NS
EPT = E // NW
CH = 100
NCH = EPT // CH
NP = 10240
RPT = NP // NS
RSTG = 80
BLK = 1024
GRID = NP // BLK

_mesh = plsc.VectorSubcoreMesh(core_axis_name="c", subcore_axis_name="s")


@functools.partial(
    pl.kernel,
    out_type=jax.ShapeDtypeStruct((NW * NP,), jnp.float32),
    mesh=_mesh,
    scratch_types=[
        pltpu.VMEM((NP,), jnp.float32),
        pltpu.VMEM((EPT,), jnp.int32),
    ],
    compiler_params=pltpu.CompilerParams(needs_layout_passes=False),
)
def _sc_degree(dst_hbm, out_hbm, acc_v, idx_v):
    c = lax.axis_index("c")
    s = lax.axis_index("s")
    wid = c * NS + s

    def zero(i, carry):
        acc_v[pl.ds(i * 16, 16)] = jnp.zeros((16,), jnp.float32)
        return carry

    lax.fori_loop(0, NP // 16, zero, 0)

    pltpu.sync_copy(dst_hbm.at[pl.ds(wid * EPT, EPT)], idx_v)
    ones = jnp.ones((16,), jnp.float32)

    def count(i, carry):
        idx = idx_v[pl.ds(i * 16, 16)]
        plsc.addupdate_scatter(acc_v, [idx], ones)
        return carry

    lax.fori_loop(0, EPT // 16, count, 0)
    pltpu.sync_copy(acc_v, out_hbm.at[pl.ds(wid * NP, NP)])


NSUP = NCH // 2


@functools.partial(
    pl.kernel,
    out_type=jax.ShapeDtypeStruct((NC, NP, D), jnp.float32),
    mesh=_mesh,
    scratch_types=[
        pltpu.VMEM((NCH, CH), jnp.int32),
        pltpu.VMEM((2, 1, CH), jnp.int32),
        pltpu.VMEM((2, CH, D), jnp.float32),
        pltpu.VMEM_SHARED((NP, D), jnp.float32),
        [pltpu.SemaphoreType.DMA] * 2,
        [pltpu.SemaphoreType.DMA] * 2,
        [pltpu.SemaphoreType.DMA] * 2,
        pltpu.SemaphoreType.DMA,
    ],
)
def _sc_scatter(hp_hbm, src4_hbm, dst3_hbm, out_hbm, didx_v, sidx_v, rows_v,
                s_sh, sem_i, sem_g, sem_s, sem_z):
    c = lax.axis_index("c")
    s = lax.axis_index("s")
    wid = c * NS + s

    pltpu.async_copy(dst3_hbm.at[wid], didx_v, sem_z)
    pltpu.async_copy(src4_hbm.at[wid, 0], sidx_v.at[0], sem_i[0])
    pltpu.async_copy(src4_hbm.at[wid, 1], sidx_v.at[1], sem_i[1])

    def zero(i, carry):
        rows_v[1, i // 8, pl.ds((i % 8) * 16, 16)] = jnp.zeros((16,), jnp.float32)
        return carry

    lax.fori_loop(0, RSTG * 8, zero, 0)
    for k in range(RPT // RSTG):
        pltpu.async_copy(rows_v.at[1, pl.ds(0, RSTG)],
                         s_sh.at[pl.ds(s * RPT + k * RSTG, RSTG)], sem_z)

    pltpu.make_async_copy(src4_hbm.at[wid, 0], sidx_v.at[0], sem_i[0]).wait()
    pltpu.async_copy(hp_hbm.at[sidx_v.at[0, 0]], rows_v.at[0], sem_g[0])
    pltpu.make_async_copy(dst3_hbm.at[wid], didx_v, sem_z).wait()
    for k in range(RPT // RSTG):
        pltpu.make_async_copy(rows_v.at[1, pl.ds(0, RSTG)],
                              s_sh.at[pl.ds(0, RSTG)], sem_z).wait()
    pltpu.make_async_copy(src4_hbm.at[wid, 1], sidx_v.at[1], sem_i[1]).wait()
    pltpu.async_copy(hp_hbm.at[sidx_v.at[1, 0]], rows_v.at[1], sem_g[1])
    plsc.subcore_barrier()

    def super_iter(si, carry):
        i0 = si * 2
        for b in range(2):
            i = i0 + b
            nxt = i + 2
            pltpu.make_async_copy(
                hp_hbm.at[sidx_v.at[b, 0]], rows_v.at[b], sem_g[b]).wait()

            @pl.when(nxt < NCH)
            def _():
                pltpu.async_copy(
                    src4_hbm.at[wid, nxt], sidx_v.at[b], sem_i[b])

            pltpu.async_copy(
                rows_v.at[b], s_sh.at[didx_v.at[i]], sem_s[b], add=True)
        for b in range(2):
            nxt = i0 + 2 + b
            pltpu.make_async_copy(
                rows_v.at[b], s_sh.at[didx_v.at[0]], sem_s[b]).wait()

            @pl.when(nxt < NCH)
            def _():
                pltpu.make_async_copy(
                    src4_hbm.at[wid, 0], sidx_v.at[b], sem_i[b]).wait()
                pltpu.async_copy(
                    hp_hbm.at[sidx_v.at[b, 0]], rows_v.at[b], sem_g[b])

        return carry

    lax.fori_loop(0, NSUP, super_iter, 0)
    plsc.subcore_barrier()

    for k in range(RPT // RSTG):
        b = k % 2
        r0 = s * RPT + k * RSTG
        if k >= 2:
            pltpu.make_async_copy(rows_v.at[b, pl.ds(0, RSTG)],
                                  out_hbm.at[c, pl.ds(0, RSTG)], sem_s[b]).wait()
        pltpu.sync_copy(s_sh.at[pl.ds(r0, RSTG)], rows_v.at[b, pl.ds(0, RSTG)])
        pltpu.async_copy(rows_v.at[b, pl.ds(0, RSTG)],
                         out_hbm.at[c, pl.ds(r0, RSTG)], sem_s[b])
    for b in range(2):
        pltpu.make_async_copy(rows_v.at[b, pl.ds(0, RSTG)],
                              out_hbm.at[c, pl.ds(0, RSTG)], sem_s[b]).wait()


def _dinv(dp_block):
    deg = jnp.sum(dp_block, axis=0) + 1.0
    return lax.rsqrt(jnp.maximum(deg, 1.0))


def _mm1_body(x_ref, w_ref, dp_ref, o_ref):
    dinv = _dinv(dp_ref[...])
    h = jnp.dot(x_ref[...], w_ref[...], preferred_element_type=jnp.float32)
    o_ref[...] = h * dinv[:, None]


_mm1 = pl.pallas_call(
    _mm1_body,
    grid=(GRID,),
    in_specs=[
        pl.BlockSpec((BLK, D), lambda i: (i, 0)),
        pl.BlockSpec((D, D), lambda i: (0, 0)),
        pl.BlockSpec((NW, BLK), lambda i: (0, i)),
    ],
    out_specs=pl.BlockSpec((BLK, D), lambda i: (i, 0)),
    out_shape=jax.ShapeDtypeStruct((N, D), jnp.float32),
)


def _mm2_body(s_ref, hp_ref, dp_ref, b_ref, w_ref, o_ref):
    dinv = _dinv(dp_ref[...])
    tot = s_ref[0] + s_ref[1] + hp_ref[...]
    z = jnp.maximum(tot * dinv[:, None] + b_ref[...], 0.0)
    h = jnp.dot(z, w_ref[...], preferred_element_type=jnp.float32)
    o_ref[...] = h * dinv[:, None]


_mm2 = pl.pallas_call(
    _mm2_body,
    grid=(GRID,),
    in_specs=[
        pl.BlockSpec((NC, BLK, D), lambda i: (0, i, 0)),
        pl.BlockSpec((BLK, D), lambda i: (i, 0)),
        pl.BlockSpec((NW, BLK), lambda i: (0, i)),
        pl.BlockSpec((1, D), lambda i: (0, 0)),
        pl.BlockSpec((D, D), lambda i: (0, 0)),
    ],
    out_specs=pl.BlockSpec((BLK, D), lambda i: (i, 0)),
    out_shape=jax.ShapeDtypeStruct((N, D), jnp.float32),
)


def _fin_body(s_ref, hp_ref, dp_ref, b_ref, o_ref):
    dinv = _dinv(dp_ref[...])
    tot = s_ref[0] + s_ref[1] + hp_ref[...]
    o_ref[...] = tot * dinv[:, None] + b_ref[...]


_fin = pl.pallas_call(
    _fin_body,
    grid=(GRID,),
    in_specs=[
        pl.BlockSpec((NC, BLK, D), lambda i: (0, i, 0)),
        pl.BlockSpec((BLK, D), lambda i: (i, 0)),
        pl.BlockSpec((NW, BLK), lambda i: (0, i)),
        pl.BlockSpec((1, D), lambda i: (0, 0)),
    ],
    out_specs=pl.BlockSpec((BLK, D), lambda i: (i, 0)),
    out_shape=jax.ShapeDtypeStruct((N, D), jnp.float32),
)


def kernel(x, edge_index, W1, b1, W2, b2):
    src = edge_index[0]
    dst = edge_index[1]
    src4 = src.reshape(NW, NCH, 1, CH)
    dst3 = dst.reshape(NW, NCH, CH)
    degp = _sc_degree(dst).reshape(NW, NP)
    h1p = _mm1(x, W1, degp)
    s1 = _sc_scatter(h1p, src4, dst3)
    h2p = _mm2(s1, h1p, degp, b1.reshape(1, D), W2)
    s2 = _sc_scatter(h2p, src4, dst3)
    return _fin(s2, h2p, degp, b2.reshape(1, D))


# speedup vs baseline: 29.7193x; 1.0646x over previous
import functools

import jax
import jax.numpy as jnp
from jax import lax
from jax.experimental import pallas as pl
from jax.experimental.pallas import tpu as pltpu
from jax.experimental.pallas import tpu_sc as plsc

N = 10000
E = 320000
D = 128

NC = 2
NS = 16
NW = NC * NS
EPT = E // NW
CH = 80
NCH = EPT // CH
NBUF = 3
NP = 10240
RPT = NP // NS
RSTG = 80
BLK = 1024
GRID = NP // BLK

_mesh = plsc.VectorSubcoreMesh(core_axis_name="c", subcore_axis_name="s")


@functools.partial(
    pl.kernel,
    out_type=jax.ShapeDtypeStruct((NW * NP,), jnp.float32),
    mesh=_mesh,
    scratch_types=[
        pltpu.VMEM((NP,), jnp.float32),
        pltpu.VMEM((EPT,), jnp.int32),
    ],
    compiler_params=pltpu.CompilerParams(needs_layout_passes=False),
)
def _sc_degree(dst_hbm, out_hbm, acc_v, idx_v):
    c = lax.axis_index("c")
    s = lax.axis_index("s")
    wid = c * NS + s

    def zero(i, carry):
        acc_v[pl.ds(i * 16, 16)] = jnp.zeros((16,), jnp.float32)
        return carry

    lax.fori_loop(0, NP // 16, zero, 0)

    pltpu.sync_copy(dst_hbm.at[pl.ds(wid * EPT, EPT)], idx_v)
    ones = jnp.ones((16,), jnp.float32)

    def count(i, carry):
        idx = idx_v[pl.ds(i * 16, 16)]
        plsc.addupdate_scatter(acc_v, [idx], ones)
        return carry

    lax.fori_loop(0, EPT // 16, count, 0)
    pltpu.sync_copy(acc_v, out_hbm.at[pl.ds(wid * NP, NP)])


NSUP = (NCH - 2) // NBUF


def _idx_load(src4_hbm, dst4_hbm, idx_v, wid, i, b, sem):
    pltpu.async_copy(src4_hbm.at[wid, i], idx_v.at[b, pl.ds(0, 1)], sem)
    pltpu.async_copy(dst4_hbm.at[wid, i], idx_v.at[b, pl.ds(1, 1)], sem)


def _idx_wait(src4_hbm, idx_v, wid, b, sem):
    for half in range(2):
        pltpu.make_async_copy(
            src4_hbm.at[wid, 0], idx_v.at[b, pl.ds(half, 1)], sem).wait()


@functools.partial(
    pl.kernel,
    out_type=jax.ShapeDtypeStruct((NC, NP, D), jnp.float32),
    mesh=_mesh,
    scratch_types=[
        pltpu.VMEM((NBUF, 2, CH), jnp.int32),
        pltpu.VMEM((NBUF, CH, D), jnp.float32),
        pltpu.VMEM_SHARED((NP, D), jnp.float32),
        [pltpu.SemaphoreType.DMA] * NBUF,
        [pltpu.SemaphoreType.DMA] * NBUF,
    ],
)
def _sc_scatter(hp_hbm, src4_hbm, dst4_hbm, out_hbm, idx_v, rows_v,
                s_sh, sem_g, sem_s):
    c = lax.axis_index("c")
    s = lax.axis_index("s")
    wid = c * NS + s
    last = NBUF - 1

    for b in range(NBUF):
        _idx_load(src4_hbm, dst4_hbm, idx_v, wid, b, b, sem_g[b])

    def zero(i, carry):
        rows_v[last, i // 8, pl.ds((i % 8) * 16, 16)] = jnp.zeros(
            (16,), jnp.float32)
        return carry

    lax.fori_loop(0, RSTG * 8, zero, 0)
    for k in range(RPT // RSTG):
        pltpu.async_copy(rows_v.at[last, pl.ds(0, RSTG)],
                         s_sh.at[pl.ds(s * RPT + k * RSTG, RSTG)], sem_s[0])

    for b in range(NBUF - 1):
        _idx_wait(src4_hbm, idx_v, wid, b, sem_g[b])
        pltpu.async_copy(hp_hbm.at[idx_v.at[b, 0]], rows_v.at[b], sem_g[b])
    for k in range(RPT // RSTG):
        pltpu.make_async_copy(rows_v.at[last, pl.ds(0, RSTG)],
                              s_sh.at[pl.ds(0, RSTG)], sem_s[0]).wait()
    _idx_wait(src4_hbm, idx_v, wid, last, sem_g[last])
    pltpu.async_copy(hp_hbm.at[idx_v.at[last, 0]], rows_v.at[last], sem_g[last])
    plsc.subcore_barrier()

    def super_iter(si, carry):
        i0 = si * NBUF
        for b in range(NBUF):
            pltpu.make_async_copy(
                hp_hbm.at[idx_v.at[b, 0]], rows_v.at[b], sem_g[b]).wait()
            pltpu.async_copy(
                rows_v.at[b], s_sh.at[idx_v.at[b, 1]], sem_s[b], add=True)
        for b in range(NBUF):
            nxt = i0 + NBUF + b
            pltpu.make_async_copy(
                rows_v.at[b], s_sh.at[idx_v.at[b, 1]], sem_s[b]).wait()

            @pl.when(nxt < NCH)
            def _():
                _idx_load(src4_hbm, dst4_hbm, idx_v, wid, nxt, b, sem_g[b])
                _idx_wait(src4_hbm, idx_v, wid, b, sem_g[b])
                pltpu.async_copy(
                    hp_hbm.at[idx_v.at[b, 0]], rows_v.at[b], sem_g[b])

        return carry

    lax.fori_loop(0, NSUP, super_iter, 0)

    for b in range(2):
        pltpu.make_async_copy(
            hp_hbm.at[idx_v.at[b, 0]], rows_v.at[b], sem_g[b]).wait()
        pltpu.async_copy(
            rows_v.at[b], s_sh.at[idx_v.at[b, 1]], sem_s[b], add=True)
    for b in range(2):
        pltpu.make_async_copy(
            rows_v.at[b], s_sh.at[idx_v.at[b, 1]], sem_s[b]).wait()
    plsc.subcore_barrier()

    for k in range(RPT // RSTG):
        b = k % NBUF
        r0 = s * RPT + k * RSTG
        if k >= NBUF:
            pltpu.make_async_copy(rows_v.at[b, pl.ds(0, RSTG)],
                                  out_hbm.at[c, pl.ds(0, RSTG)], sem_s[b]).wait()
        pltpu.sync_copy(s_sh.at[pl.ds(r0, RSTG)], rows_v.at[b, pl.ds(0, RSTG)])
        pltpu.async_copy(rows_v.at[b, pl.ds(0, RSTG)],
                         out_hbm.at[c, pl.ds(r0, RSTG)], sem_s[b])
    for b in range(NBUF):
        pltpu.make_async_copy(rows_v.at[b, pl.ds(0, RSTG)],
                              out_hbm.at[c, pl.ds(0, RSTG)], sem_s[b]).wait()


def _dinv(dp_block):
    deg = jnp.sum(dp_block, axis=0) + 1.0
    return lax.rsqrt(jnp.maximum(deg, 1.0))


def _mm1_body(x_ref, w_ref, dp_ref, o_ref):
    dinv = _dinv(dp_ref[...])
    h = jnp.dot(x_ref[...], w_ref[...], preferred_element_type=jnp.float32)
    o_ref[...] = h * dinv[:, None]


_mm1 = pl.pallas_call(
    _mm1_body,
    grid=(GRID,),
    in_specs=[
        pl.BlockSpec((BLK, D), lambda i: (i, 0)),
        pl.BlockSpec((D, D), lambda i: (0, 0)),
        pl.BlockSpec((NW, BLK), lambda i: (0, i)),
    ],
    out_specs=pl.BlockSpec((BLK, D), lambda i: (i, 0)),
    out_shape=jax.ShapeDtypeStruct((N, D), jnp.float32),
)


def _mm2_body(s_ref, hp_ref, dp_ref, b_ref, w_ref, o_ref):
    dinv = _dinv(dp_ref[...])
    tot = s_ref[0] + s_ref[1] + hp_ref[...]
    z = jnp.maximum(tot * dinv[:, None] + b_ref[...], 0.0)
    h = jnp.dot(z, w_ref[...], preferred_element_type=jnp.float32)
    o_ref[...] = h * dinv[:, None]


_mm2 = pl.pallas_call(
    _mm2_body,
    grid=(GRID,),
    in_specs=[
        pl.BlockSpec((NC, BLK, D), lambda i: (0, i, 0)),
        pl.BlockSpec((BLK, D), lambda i: (i, 0)),
        pl.BlockSpec((NW, BLK), lambda i: (0, i)),
        pl.BlockSpec((1, D), lambda i: (0, 0)),
        pl.BlockSpec((D, D), lambda i: (0, 0)),
    ],
    out_specs=pl.BlockSpec((BLK, D), lambda i: (i, 0)),
    out_shape=jax.ShapeDtypeStruct((N, D), jnp.float32),
)


def _fin_body(s_ref, hp_ref, dp_ref, b_ref, o_ref):
    dinv = _dinv(dp_ref[...])
    tot = s_ref[0] + s_ref[1] + hp_ref[...]
    o_ref[...] = tot * dinv[:, None] + b_ref[...]


_fin = pl.pallas_call(
    _fin_body,
    grid=(GRID,),
    in_specs=[
        pl.BlockSpec((NC, BLK, D), lambda i: (0, i, 0)),
        pl.BlockSpec((BLK, D), lambda i: (i, 0)),
        pl.BlockSpec((NW, BLK), lambda i: (0, i)),
        pl.BlockSpec((1, D), lambda i: (0, 0)),
    ],
    out_specs=pl.BlockSpec((BLK, D), lambda i: (i, 0)),
    out_shape=jax.ShapeDtypeStruct((N, D), jnp.float32),
)


def kernel(x, edge_index, W1, b1, W2, b2):
    src = edge_index[0]
    dst = edge_index[1]
    src4 = src.reshape(NW, NCH, 1, CH)
    dst4 = dst.reshape(NW, NCH, 1, CH)
    degp = _sc_degree(dst).reshape(NW, NP)
    h1p = _mm1(x, W1, degp)
    s1 = _sc_scatter(h1p, src4, dst4)
    h2p = _mm2(s1, h1p, degp, b1.reshape(1, D), W2)
    s2 = _sc_scatter(h2p, src4, dst4)
    return _fin(s2, h2p, degp, b2.reshape(1, D))


# speedup vs baseline: 31.3940x; 1.0564x over previous
import functools

import jax
import jax.numpy as jnp
from jax import lax
from jax.experimental import pallas as pl
from jax.experimental.pallas import tpu as pltpu
from jax.experimental.pallas import tpu_sc as plsc

N = 10000
E = 320000
D = 128

NC = 2
NS = 16
NW = NC * NS
EPT = E // NW
CH = 80
NCH = EPT // CH
NBUF = 3
NP = 10240
RPT = NP // NS
RSTG = 80
BLK = 1024
GRID = NP // BLK

_mesh = plsc.VectorSubcoreMesh(core_axis_name="c", subcore_axis_name="s")


@functools.partial(
    pl.kernel,
    out_type=jax.ShapeDtypeStruct((NW * NP,), jnp.float32),
    mesh=_mesh,
    scratch_types=[
        pltpu.VMEM((NP,), jnp.float32),
        pltpu.VMEM((EPT,), jnp.int32),
    ],
    compiler_params=pltpu.CompilerParams(needs_layout_passes=False),
)
def _sc_degree(dst_hbm, out_hbm, acc_v, idx_v):
    c = lax.axis_index("c")
    s = lax.axis_index("s")
    wid = c * NS + s

    def zero(i, carry):
        acc_v[pl.ds(i * 16, 16)] = jnp.zeros((16,), jnp.float32)
        return carry

    lax.fori_loop(0, NP // 16, zero, 0)

    pltpu.sync_copy(dst_hbm.at[pl.ds(wid * EPT, EPT)], idx_v)
    ones = jnp.ones((16,), jnp.float32)

    def count(i, carry):
        idx = idx_v[pl.ds(i * 16, 16)]
        plsc.addupdate_scatter(acc_v, [idx], ones)
        return carry

    lax.fori_loop(0, EPT // 16, count, 0)
    pltpu.sync_copy(acc_v, out_hbm.at[pl.ds(wid * NP, NP)])


NSUP = (NCH - 2) // NBUF


def _half_load(arr4_hbm, idx_v, wid, i, b, half, sem):
    pltpu.async_copy(arr4_hbm.at[wid, i], idx_v.at[b, pl.ds(half, 1)], sem)


def _half_wait(arr4_hbm, idx_v, wid, b, half, sem):
    pltpu.make_async_copy(
        arr4_hbm.at[wid, 0], idx_v.at[b, pl.ds(half, 1)], sem).wait()


@functools.partial(
    pl.kernel,
    out_type=jax.ShapeDtypeStruct((NC, NP, D), jnp.float32),
    mesh=_mesh,
    scratch_types=[
        pltpu.VMEM((NBUF, 2, CH), jnp.int32),
        pltpu.VMEM((NBUF, CH, D), jnp.float32),
        pltpu.VMEM_SHARED((NP, D), jnp.float32),
        [pltpu.SemaphoreType.DMA] * NBUF,
        [pltpu.SemaphoreType.DMA] * NBUF,
    ],
)
def _sc_scatter(hp_hbm, src4_hbm, dst4_hbm, out_hbm, idx_v, rows_v,
                s_sh, sem_g, sem_s):
    c = lax.axis_index("c")
    s = lax.axis_index("s")
    wid = c * NS + s
    last = NBUF - 1

    for b in range(NBUF):
        _half_load(src4_hbm, idx_v, wid, b, b, 0, sem_g[b])
        _half_load(dst4_hbm, idx_v, wid, b, b, 1, sem_s[b])

    def zero(i, carry):
        rows_v[last, i // 8, pl.ds((i % 8) * 16, 16)] = jnp.zeros(
            (16,), jnp.float32)
        return carry

    lax.fori_loop(0, RSTG * 8, zero, 0)
    for k in range(RPT // RSTG):
        pltpu.async_copy(rows_v.at[last, pl.ds(0, RSTG)],
                         s_sh.at[pl.ds(s * RPT + k * RSTG, RSTG)], sem_s[0])

    for b in range(NBUF - 1):
        _half_wait(src4_hbm, idx_v, wid, b, 0, sem_g[b])
        pltpu.async_copy(hp_hbm.at[idx_v.at[b, 0]], rows_v.at[b], sem_g[b])
    for k in range(RPT // RSTG):
        pltpu.make_async_copy(rows_v.at[last, pl.ds(0, RSTG)],
                              s_sh.at[pl.ds(0, RSTG)], sem_s[0]).wait()
    _half_wait(src4_hbm, idx_v, wid, last, 0, sem_g[last])
    pltpu.async_copy(hp_hbm.at[idx_v.at[last, 0]], rows_v.at[last], sem_g[last])
    plsc.subcore_barrier()

    def super_iter(si, carry):
        i0 = si * NBUF
        for b in range(NBUF):
            nxt = i0 + b + NBUF
            pltpu.make_async_copy(
                hp_hbm.at[idx_v.at[b, 0]], rows_v.at[b], sem_g[b]).wait()

            @pl.when(nxt < NCH)
            def _():
                _half_load(src4_hbm, idx_v, wid, nxt, b, 0, sem_g[b])

            _half_wait(dst4_hbm, idx_v, wid, b, 1, sem_s[b])
            pltpu.async_copy(
                rows_v.at[b], s_sh.at[idx_v.at[b, 1]], sem_s[b], add=True)
        for b in range(NBUF):
            nxt = i0 + NBUF + b
            pltpu.make_async_copy(
                rows_v.at[b], s_sh.at[idx_v.at[b, 1]], sem_s[b]).wait()

            @pl.when(nxt < NCH)
            def _():
                _half_load(dst4_hbm, idx_v, wid, nxt, b, 1, sem_s[b])
                _half_wait(src4_hbm, idx_v, wid, b, 0, sem_g[b])
                pltpu.async_copy(
                    hp_hbm.at[idx_v.at[b, 0]], rows_v.at[b], sem_g[b])

        return carry

    lax.fori_loop(0, NSUP, super_iter, 0)

    for b in range(2):
        pltpu.make_async_copy(
            hp_hbm.at[idx_v.at[b, 0]], rows_v.at[b], sem_g[b]).wait()
        _half_wait(dst4_hbm, idx_v, wid, b, 1, sem_s[b])
        pltpu.async_copy(
            rows_v.at[b], s_sh.at[idx_v.at[b, 1]], sem_s[b], add=True)
    for b in range(2):
        pltpu.make_async_copy(
            rows_v.at[b], s_sh.at[idx_v.at[b, 1]], sem_s[b]).wait()
    plsc.subcore_barrier()

    for k in range(RPT // RSTG):
        b = k % NBUF
        r0 = s * RPT + k * RSTG
        if k >= NBUF:
            pltpu.make_async_copy(rows_v.at[b, pl.ds(0, RSTG)],
                                  out_hbm.at[c, pl.ds(0, RSTG)], sem_s[b]).wait()
        pltpu.sync_copy(s_sh.at[pl.ds(r0, RSTG)], rows_v.at[b, pl.ds(0, RSTG)])
        pltpu.async_copy(rows_v.at[b, pl.ds(0, RSTG)],
                         out_hbm.at[c, pl.ds(r0, RSTG)], sem_s[b])
    for b in range(NBUF):
        pltpu.make_async_copy(rows_v.at[b, pl.ds(0, RSTG)],
                              out_hbm.at[c, pl.ds(0, RSTG)], sem_s[b]).wait()


def _dinv(dp_block):
    deg = jnp.sum(dp_block, axis=0) + 1.0
    return lax.rsqrt(jnp.maximum(deg, 1.0))


def _mm1_body(x_ref, w_ref, dp_ref, o_ref):
    dinv = _dinv(dp_ref[...])
    h = jnp.dot(x_ref[...], w_ref[...], preferred_element_type=jnp.float32)
    o_ref[...] = h * dinv[:, None]


_mm1 = pl.pallas_call(
    _mm1_body,
    grid=(GRID,),
    in_specs=[
        pl.BlockSpec((BLK, D), lambda i: (i, 0)),
        pl.BlockSpec((D, D), lambda i: (0, 0)),
        pl.BlockSpec((NW, BLK), lambda i: (0, i)),
    ],
    out_specs=pl.BlockSpec((BLK, D), lambda i: (i, 0)),
    out_shape=jax.ShapeDtypeStruct((N, D), jnp.float32),
)


def _mm2_body(s_ref, hp_ref, dp_ref, b_ref, w_ref, o_ref):
    dinv = _dinv(dp_ref[...])
    tot = s_ref[0] + s_ref[1] + hp_ref[...]
    z = jnp.maximum(tot * dinv[:, None] + b_ref[...], 0.0)
    h = jnp.dot(z, w_ref[...], preferred_element_type=jnp.float32)
    o_ref[...] = h * dinv[:, None]


_mm2 = pl.pallas_call(
    _mm2_body,
    grid=(GRID,),
    in_specs=[
        pl.BlockSpec((NC, BLK, D), lambda i: (0, i, 0)),
        pl.BlockSpec((BLK, D), lambda i: (i, 0)),
        pl.BlockSpec((NW, BLK), lambda i: (0, i)),
        pl.BlockSpec((1, D), lambda i: (0, 0)),
        pl.BlockSpec((D, D), lambda i: (0, 0)),
    ],
    out_specs=pl.BlockSpec((BLK, D), lambda i: (i, 0)),
    out_shape=jax.ShapeDtypeStruct((N, D), jnp.float32),
)


def _fin_body(s_ref, hp_ref, dp_ref, b_ref, o_ref):
    dinv = _dinv(dp_ref[...])
    tot = s_ref[0] + s_ref[1] + hp_ref[...]
    o_ref[...] = tot * dinv[:, None] + b_ref[...]


_fin = pl.pallas_call(
    _fin_body,
    grid=(GRID,),
    in_specs=[
        pl.BlockSpec((NC, BLK, D), lambda i: (0, i, 0)),
        pl.BlockSpec((BLK, D), lambda i: (i, 0)),
        pl.BlockSpec((NW, BLK), lambda i: (0, i)),
        pl.BlockSpec((1, D), lambda i: (0, 0)),
    ],
    out_specs=pl.BlockSpec((BLK, D), lambda i: (i, 0)),
    out_shape=jax.ShapeDtypeStruct((N, D), jnp.float32),
)


def kernel(x, edge_index, W1, b1, W2, b2):
    src = edge_index[0]
    dst = edge_index[1]
    src4 = src.reshape(NW, NCH, 1, CH)
    dst4 = dst.reshape(NW, NCH, 1, CH)
    degp = _sc_degree(dst).reshape(NW, NP)
    h1p = _mm1(x, W1, degp)
    s1 = _sc_scatter(h1p, src4, dst4)
    h2p = _mm2(s1, h1p, degp, b1.reshape(1, D), W2)
    s2 = _sc_scatter(h2p, src4, dst4)
    return _fin(s2, h2p, degp, b2.reshape(1, D))


# speedup vs baseline: 33.6931x; 1.0732x over previous
import functools

import jax
import jax.numpy as jnp
from jax import lax
from jax.experimental import pallas as pl
from jax.experimental.pallas import tpu as pltpu
from jax.experimental.pallas import tpu_sc as plsc

N = 10000
E = 320000
D = 128

NC = 2
NS = 16
NW = NC * NS
EPT = E // NW
CH = 80
NCH = EPT // CH
NBUF = 4
NP = 10240
RPT = NP // NS
RSTG = 80
BLK = 1024
GRID = NP // BLK

_mesh = plsc.VectorSubcoreMesh(core_axis_name="c", subcore_axis_name="s")


@functools.partial(
    pl.kernel,
    out_type=jax.ShapeDtypeStruct((NW * NP,), jnp.float32),
    mesh=_mesh,
    scratch_types=[
        pltpu.VMEM((NP,), jnp.float32),
        pltpu.VMEM((EPT,), jnp.int32),
    ],
    compiler_params=pltpu.CompilerParams(needs_layout_passes=False),
)
def _sc_degree(dst_hbm, out_hbm, acc_v, idx_v):
    c = lax.axis_index("c")
    s = lax.axis_index("s")
    wid = c * NS + s

    def zero(i, carry):
        acc_v[pl.ds(i * 16, 16)] = jnp.zeros((16,), jnp.float32)
        return carry

    lax.fori_loop(0, NP // 16, zero, 0)

    pltpu.sync_copy(dst_hbm.at[pl.ds(wid * EPT, EPT)], idx_v)
    ones = jnp.ones((16,), jnp.float32)

    def count(i, carry):
        idx = idx_v[pl.ds(i * 16, 16)]
        plsc.addupdate_scatter(acc_v, [idx], ones)
        return carry

    lax.fori_loop(0, EPT // 16, count, 0)
    pltpu.sync_copy(acc_v, out_hbm.at[pl.ds(wid * NP, NP)])


NSUP = (NCH - 1) // NBUF


def _half_load(arr4_hbm, idx_v, wid, i, b, half, sem):
    pltpu.async_copy(arr4_hbm.at[wid, i], idx_v.at[b, pl.ds(half, 1)], sem)


def _half_wait(arr4_hbm, idx_v, wid, b, half, sem):
    pltpu.make_async_copy(
        arr4_hbm.at[wid, 0], idx_v.at[b, pl.ds(half, 1)], sem).wait()


@functools.partial(
    pl.kernel,
    out_type=jax.ShapeDtypeStruct((NC, NP, D), jnp.float32),
    mesh=_mesh,
    scratch_types=[
        pltpu.VMEM((NBUF, 2, CH), jnp.int32),
        pltpu.VMEM((NBUF, CH, D), jnp.float32),
        pltpu.VMEM_SHARED((NP, D), jnp.float32),
        [pltpu.SemaphoreType.DMA] * NBUF,
        [pltpu.SemaphoreType.DMA] * NBUF,
    ],
)
def _sc_scatter(hp_hbm, src4_hbm, dst4_hbm, out_hbm, idx_v, rows_v,
                s_sh, sem_g, sem_s):
    c = lax.axis_index("c")
    s = lax.axis_index("s")
    wid = c * NS + s
    last = NBUF - 1

    for b in range(NBUF):
        _half_load(src4_hbm, idx_v, wid, b, b, 0, sem_g[b])
        _half_load(dst4_hbm, idx_v, wid, b, b, 1, sem_s[b])

    def zero(i, carry):
        rows_v[last, i // 8, pl.ds((i % 8) * 16, 16)] = jnp.zeros(
            (16,), jnp.float32)
        return carry

    lax.fori_loop(0, RSTG * 8, zero, 0)
    for k in range(RPT // RSTG):
        pltpu.async_copy(rows_v.at[last, pl.ds(0, RSTG)],
                         s_sh.at[pl.ds(s * RPT + k * RSTG, RSTG)], sem_s[0])

    for b in range(NBUF - 1):
        _half_wait(src4_hbm, idx_v, wid, b, 0, sem_g[b])
        pltpu.async_copy(hp_hbm.at[idx_v.at[b, 0]], rows_v.at[b], sem_g[b])
    for k in range(RPT // RSTG):
        pltpu.make_async_copy(rows_v.at[last, pl.ds(0, RSTG)],
                              s_sh.at[pl.ds(0, RSTG)], sem_s[0]).wait()
    _half_wait(src4_hbm, idx_v, wid, last, 0, sem_g[last])
    pltpu.async_copy(hp_hbm.at[idx_v.at[last, 0]], rows_v.at[last], sem_g[last])
    plsc.subcore_barrier()

    def super_iter(si, carry):
        i0 = si * NBUF
        for b in range(NBUF):
            nxt = i0 + b + NBUF
            pltpu.make_async_copy(
                hp_hbm.at[idx_v.at[b, 0]], rows_v.at[b], sem_g[b]).wait()

            @pl.when(nxt < NCH)
            def _():
                _half_load(src4_hbm, idx_v, wid, nxt, b, 0, sem_g[b])

            _half_wait(dst4_hbm, idx_v, wid, b, 1, sem_s[b])
            pltpu.async_copy(
                rows_v.at[b], s_sh.at[idx_v.at[b, 1]], sem_s[b], add=True)
        for b in range(NBUF):
            nxt = i0 + NBUF + b
            pltpu.make_async_copy(
                rows_v.at[b], s_sh.at[idx_v.at[b, 1]], sem_s[b]).wait()

            @pl.when(nxt < NCH)
            def _():
                _half_load(dst4_hbm, idx_v, wid, nxt, b, 1, sem_s[b])
                _half_wait(src4_hbm, idx_v, wid, b, 0, sem_g[b])
                pltpu.async_copy(
                    hp_hbm.at[idx_v.at[b, 0]], rows_v.at[b], sem_g[b])

        return carry

    lax.fori_loop(0, NSUP, super_iter, 0)

    for b in range(1):
        pltpu.make_async_copy(
            hp_hbm.at[idx_v.at[b, 0]], rows_v.at[b], sem_g[b]).wait()
        _half_wait(dst4_hbm, idx_v, wid, b, 1, sem_s[b])
        pltpu.async_copy(
            rows_v.at[b], s_sh.at[idx_v.at[b, 1]], sem_s[b], add=True)
    for b in range(1):
        pltpu.make_async_copy(
            rows_v.at[b], s_sh.at[idx_v.at[b, 1]], sem_s[b]).wait()
    plsc.subcore_barrier()

    for k in range(RPT // RSTG):
        b = k % NBUF
        r0 = s * RPT + k * RSTG
        if k >= NBUF:
            pltpu.make_async_copy(rows_v.at[b, pl.ds(0, RSTG)],
                                  out_hbm.at[c, pl.ds(0, RSTG)], sem_s[b]).wait()
        pltpu.sync_copy(s_sh.at[pl.ds(r0, RSTG)], rows_v.at[b, pl.ds(0, RSTG)])
        pltpu.async_copy(rows_v.at[b, pl.ds(0, RSTG)],
                         out_hbm.at[c, pl.ds(r0, RSTG)], sem_s[b])
    for b in range(NBUF):
        pltpu.make_async_copy(rows_v.at[b, pl.ds(0, RSTG)],
                              out_hbm.at[c, pl.ds(0, RSTG)], sem_s[b]).wait()


def _dinv(dp_block):
    deg = jnp.sum(dp_block, axis=0) + 1.0
    return lax.rsqrt(jnp.maximum(deg, 1.0))


def _mm1_body(x_ref, w_ref, dp_ref, o_ref):
    dinv = _dinv(dp_ref[...])
    h = jnp.dot(x_ref[...], w_ref[...], preferred_element_type=jnp.float32)
    o_ref[...] = h * dinv[:, None]


_mm1 = pl.pallas_call(
    _mm1_body,
    grid=(GRID,),
    in_specs=[
        pl.BlockSpec((BLK, D), lambda i: (i, 0)),
        pl.BlockSpec((D, D), lambda i: (0, 0)),
        pl.BlockSpec((NW, BLK), lambda i: (0, i)),
    ],
    out_specs=pl.BlockSpec((BLK, D), lambda i: (i, 0)),
    out_shape=jax.ShapeDtypeStruct((N, D), jnp.float32),
)


def _mm2_body(s_ref, hp_ref, dp_ref, b_ref, w_ref, o_ref):
    dinv = _dinv(dp_ref[...])
    tot = s_ref[0] + s_ref[1] + hp_ref[...]
    z = jnp.maximum(tot * dinv[:, None] + b_ref[...], 0.0)
    h = jnp.dot(z, w_ref[...], preferred_element_type=jnp.float32)
    o_ref[...] = h * dinv[:, None]


_mm2 = pl.pallas_call(
    _mm2_body,
    grid=(GRID,),
    in_specs=[
        pl.BlockSpec((NC, BLK, D), lambda i: (0, i, 0)),
        pl.BlockSpec((BLK, D), lambda i: (i, 0)),
        pl.BlockSpec((NW, BLK), lambda i: (0, i)),
        pl.BlockSpec((1, D), lambda i: (0, 0)),
        pl.BlockSpec((D, D), lambda i: (0, 0)),
    ],
    out_specs=pl.BlockSpec((BLK, D), lambda i: (i, 0)),
    out_shape=jax.ShapeDtypeStruct((N, D), jnp.float32),
)


def _fin_body(s_ref, hp_ref, dp_ref, b_ref, o_ref):
    dinv = _dinv(dp_ref[...])
    tot = s_ref[0] + s_ref[1] + hp_ref[...]
    o_ref[...] = tot * dinv[:, None] + b_ref[...]


_fin = pl.pallas_call(
    _fin_body,
    grid=(GRID,),
    in_specs=[
        pl.BlockSpec((NC, BLK, D), lambda i: (0, i, 0)),
        pl.BlockSpec((BLK, D), lambda i: (i, 0)),
        pl.BlockSpec((NW, BLK), lambda i: (0, i)),
        pl.BlockSpec((1, D), lambda i: (0, 0)),
    ],
    out_specs=pl.BlockSpec((BLK, D), lambda i: (i, 0)),
    out_shape=jax.ShapeDtypeStruct((N, D), jnp.float32),
)


def kernel(x, edge_index, W1, b1, W2, b2):
    src = edge_index[0]
    dst = edge_index[1]
    src4 = src.reshape(NW, NCH, 1, CH)
    dst4 = dst.reshape(NW, NCH, 1, CH)
    degp = _sc_degree(dst).reshape(NW, NP)
    h1p = _mm1(x, W1, degp)
    s1 = _sc_scatter(h1p, src4, dst4)
    h2p = _mm2(s1, h1p, degp, b1.reshape(1, D), W2)
    s2 = _sc_scatter(h2p, src4, dst4)
    return _fin(s2, h2p, degp, b2.reshape(1, D))
